# baseline (speedup 1.0000x reference)
import functools

import jax
import jax.numpy as jnp
from jax import lax
from jax.experimental import pallas as pl
from jax.experimental.pallas import tpu as pltpu
from jax.experimental.pallas import tpu_sc as plsc

_N = 10000
_E = 320000
_D = 128
_NC = 2
_NS = 16
_NW = _NC * _NS
_CHUNK = 80
_NCHUNK = 128
_EPW = _NCHUNK * _CHUNK
_EPAD = _NW * _EPW - _E
_RPT = 1000
_ZROWS = 200
_CNTC = 1000
_NBUF = 2


_CW = 10016


def _make_sc_agg(with_cnt: bool):
    mesh = plsc.VectorSubcoreMesh(core_axis_name="c", subcore_axis_name="s",
                                  num_cores=_NC, num_subcores=_NS)
    out_type = [jax.ShapeDtypeStruct((_NC, _N, _D), jnp.float32)]
    if with_cnt:
        out_type.append(jax.ShapeDtypeStruct((_NW * _CW,), jnp.float32))
    _H0 = _NCHUNK // 2
    scratch = (
        [pltpu.VMEM((_CHUNK,), jnp.int32)] * _NBUF
        + [pltpu.VMEM((_H0, _CHUNK), jnp.int32)]
        + [pltpu.VMEM((_CHUNK, _D), jnp.float32)] * _NBUF
        + ([pltpu.VMEM((_CW,), jnp.float32)] if with_cnt else [])
        + [pltpu.VMEM_SHARED((_N + 8, _D), jnp.float32)]
        + [pltpu.SemaphoreType.DMA] * (3 * _NBUF + 1)
    )

    def body(src_hbm, dst_hbm, x_hbm, *refs):
        if with_cnt:
            agg_out, cnt_out = refs[0], refs[1]
            rest = refs[2:]
        else:
            agg_out = refs[0]
            rest = refs[1:]
        srcc = rest[0:_NBUF]
        dst_v = rest[_NBUF]
        rows = rest[_NBUF + 1:2 * _NBUF + 1]
        rest = rest[2 * _NBUF + 1:]
        if with_cnt:
            cnt_v = rest[0]
            rest = rest[1:]
        agg_sh = rest[0]
        sems = rest[1:]
        isem = sems[0:_NBUF]
        gsem = sems[_NBUF:2 * _NBUF]
        ssem = sems[2 * _NBUF:3 * _NBUF]
        dsem = sems[3 * _NBUF]

        cid = lax.axis_index("c")
        sid = lax.axis_index("s")
        wid = sid * _NC + cid
        base0 = wid * _EPW

        def dst_load(h):
            return pltpu.async_copy(
                dst_hbm.at[wid, pl.ds(h * _H0, _H0)], dst_v, dsem)

        def src_load(c, j):
            off = pl.multiple_of(base0 + c * _CHUNK, 8)
            pltpu.async_copy(src_hbm.at[pl.ds(off, _CHUNK)], srcc[j], isem[j])

        def src_wait(j):
            pltpu.make_async_copy(src_hbm.at[pl.ds(0, _CHUNK)], srcc[j],
                                  isem[j]).wait()

        def gather_start(c, b):
            pltpu.async_copy(x_hbm.at[srcc[b]], rows[b], gsem[b])

        def gather_wait(b):
            pltpu.make_async_copy(x_hbm.at[srcc[b]], rows[b], gsem[b]).wait()

        def scatter_start(lc, b):
            pltpu.async_copy(rows[b], agg_sh.at[dst_v.at[lc]], ssem[b],
                             add=True)

        def scatter_wait(b):
            pltpu.make_async_copy(rows[b], agg_sh.at[dst_v.at[0]],
                                  ssem[b]).wait()

        dst_cp = dst_load(0)
        src_load(0, 0)
        z16 = jnp.zeros((16,), jnp.float32)

        def fill_zrow(i, _):
            r = i // 8
            col = (i % 8) * 16
            rows[0][r, pl.ds(col, 16)] = z16
            return 0

        lax.fori_loop(0, _CHUNK * 8, fill_zrow, 0)

        if with_cnt:
            def fill_zcnt(i, _):
                cnt_v[pl.ds(i * 16, 16)] = z16
                return 0

            lax.fori_loop(0, _CW // 16, fill_zcnt, 0)

        _TAIL = _RPT - (_RPT // _CHUNK) * _CHUNK

        @pl.when(sid < _N // _RPT)
        def _():
            zcp = []
            for k in range(_RPT // _CHUNK):
                zcp.append(pltpu.async_copy(
                    rows[0], agg_sh.at[pl.ds(sid * _RPT + k * _CHUNK, _CHUNK)],
                    ssem[0]))
            zcp.append(pltpu.async_copy(
                rows[0].at[pl.ds(0, _TAIL)],
                agg_sh.at[pl.ds(sid * _RPT + (_RPT // _CHUNK) * _CHUNK,
                                _TAIL)], ssem[0]))
            for cp in zcp:
                cp.wait()
        @pl.when(sid == _N // _RPT)
        def _():
            pltpu.async_copy(rows[0].at[pl.ds(0, 8)],
                             agg_sh.at[pl.ds(_N, 8)], ssem[0]).wait()
        dst_cp.wait()
        plsc.subcore_barrier()

        one16 = jnp.ones((16,), jnp.float32)

        def count_half():
            if not with_cnt:
                return

            def cl(l, _):
                for j in range(_CHUNK // 16):
                    dvec = dst_v[l, pl.ds(j * 16, 16)]
                    plsc.addupdate_scatter(cnt_v, [dvec], one16)
                return 0

            lax.fori_loop(0, _H0, cl, 0)

        def run_half(c0):
            for b in range(_NBUF):
                src_wait(b)
                gather_start(c0 + b, b)
                if b >= 1:
                    gather_wait(b - 1)
                    src_load(c0 + b + 1, (b + 1) % _NBUF)
                    scatter_start(b - 1, b - 1)
                elif _NBUF > 1:
                    src_load(c0 + 1, 1)

            def group(g, _):
                for b in range(_NBUF):
                    l = g * _NBUF + b
                    bp = (b - 1) % _NBUF
                    bn = (b + 1) % _NBUF
                    scatter_wait(b)
                    src_wait(b)
                    gather_start(c0 + l, b)
                    gather_wait(bp)
                    @pl.when(c0 + l + 1 < _NCHUNK)
                    def _():
                        src_load(c0 + l + 1, bn)
                    scatter_start(l - 1, bp)
                return 0

            lax.fori_loop(1, _H0 // _NBUF, group, 0)

            blast = (_H0 - 1) % _NBUF
            gather_wait(blast)
            scatter_start(_H0 - 1, blast)
            for b in range(_NBUF):
                scatter_wait(b)

        run_half(0)
        count_half()
        dst_load(1).wait()
        run_half(_H0)
        count_half()

        plsc.subcore_barrier()

        ocp = []
        if with_cnt:
            ocp.append(pltpu.async_copy(
                cnt_v, cnt_out.at[pl.ds(wid * _CW, _CW)], ssem[0]))

        @pl.when(sid < _N // _RPT)
        def _():
            ocp2 = []
            for k in range(_RPT // _ZROWS):
                rs = sid * _RPT + k * _ZROWS
                ocp2.append(pltpu.async_copy(agg_sh.at[pl.ds(rs, _ZROWS)],
                                             agg_out.at[cid,
                                                        pl.ds(rs, _ZROWS)],
                                             ssem[1]))
            for cp in ocp2:
                cp.wait()
        for cp in ocp:
            cp.wait()

    return pl.kernel(body, out_type=out_type, mesh=mesh, scratch_types=scratch,
                     name="sc_sage_agg_cnt" if with_cnt else "sc_sage_agg")


_make_sc_agg = functools.lru_cache(maxsize=None)(_make_sc_agg)

_BM = 1000


def _make_tc_skip():
    def body(x_ref, wr_ref, b_ref, o_ref):
        o_ref[...] = (jnp.dot(x_ref[...], wr_ref[...],
                              preferred_element_type=jnp.float32)
                      + b_ref[...])

    return pl.pallas_call(
        body,
        grid=(_N // _BM,),
        in_specs=[
            pl.BlockSpec((_BM, _D), lambda i: (i, 0)),
            pl.BlockSpec((_D, _D), lambda i: (0, 0)),
            pl.BlockSpec((1, _D), lambda i: (0, 0)),
        ],
        out_specs=pl.BlockSpec((_BM, _D), lambda i: (i, 0)),
        out_shape=jax.ShapeDtypeStruct((_N, _D), jnp.float32),
        name="tc_sage_skip",
    )


def _make_tc_combine1():
    def body(agg_ref, crec_ref, xr_ref, wl_ref, wr2_ref, b2_ref, h_ref,
             xr2_ref):
        a = agg_ref[0] + agg_ref[1]
        mean = a * crec_ref[...]
        h = jnp.maximum(
            jnp.dot(mean, wl_ref[...], preferred_element_type=jnp.float32)
            + xr_ref[...], 0.0)
        h_ref[...] = h
        xr2_ref[...] = (jnp.dot(h, wr2_ref[...],
                                preferred_element_type=jnp.float32)
                        + b2_ref[...])

    return pl.pallas_call(
        body,
        grid=(_N // _BM,),
        in_specs=[
            pl.BlockSpec((_NC, _BM, _D), lambda i: (0, i, 0)),
            pl.BlockSpec((_BM, 1), lambda i: (i, 0)),
            pl.BlockSpec((_BM, _D), lambda i: (i, 0)),
            pl.BlockSpec((_D, _D), lambda i: (0, 0)),
            pl.BlockSpec((_D, _D), lambda i: (0, 0)),
            pl.BlockSpec((1, _D), lambda i: (0, 0)),
        ],
        out_specs=[pl.BlockSpec((_BM, _D), lambda i: (i, 0)),
                   pl.BlockSpec((_BM, _D), lambda i: (i, 0))],
        out_shape=[jax.ShapeDtypeStruct((_N, _D), jnp.float32),
                   jax.ShapeDtypeStruct((_N, _D), jnp.float32)],
        name="tc_sage_combine1",
    )


def _make_tc_combine2():
    def body(agg_ref, crec_ref, xr_ref, wl_ref, o_ref):
        a = agg_ref[0] + agg_ref[1]
        mean = a * crec_ref[...]
        o_ref[...] = jnp.dot(mean, wl_ref[...],
                             preferred_element_type=jnp.float32) + xr_ref[...]

    return pl.pallas_call(
        body,
        grid=(_N // _BM,),
        in_specs=[
            pl.BlockSpec((_NC, _BM, _D), lambda i: (0, i, 0)),
            pl.BlockSpec((_BM, 1), lambda i: (i, 0)),
            pl.BlockSpec((_BM, _D), lambda i: (i, 0)),
            pl.BlockSpec((_D, _D), lambda i: (0, 0)),
        ],
        out_specs=pl.BlockSpec((_BM, _D), lambda i: (i, 0)),
        out_shape=jax.ShapeDtypeStruct((_N, _D), jnp.float32),
        name="tc_sage_combine2",
    )


_tc_skip = _make_tc_skip()
_tc_combine1 = _make_tc_combine1()
_tc_combine2 = _make_tc_combine2()


@jax.jit
def kernel(x, edge_index, W_l1, W_r1, b1, W_l2, W_r2, b2):
    src = jnp.pad(edge_index[0].astype(jnp.int32), (0, _EPAD + 320))
    dst = jnp.pad(edge_index[1].astype(jnp.int32), (0, _EPAD),
                  constant_values=_N)[: _NW * _EPW]
    dst = dst.reshape(_NW, _NCHUNK, _CHUNK)

    xr1 = _tc_skip(x, W_r1, b1.reshape(1, _D))
    (agg1,) = _make_sc_agg(False)(src, dst, x)
    cnt_tot = jax.ops.segment_sum(jnp.ones((_E,), jnp.float32),
                                  edge_index[1], num_segments=_N)
    crec = (1.0 / jnp.maximum(cnt_tot, 1.0)).reshape(_N, 1)
    h, xr2 = _tc_combine1(agg1, crec, xr1, W_l1, W_r2, b2.reshape(1, _D))
    (agg2,) = _make_sc_agg(False)(src, dst, h)
    out = _tc_combine2(agg2, crec, xr2, W_l2)
    return out


# speedup vs baseline: 1.2995x; 1.2995x over previous
import functools

import jax
import jax.numpy as jnp
from jax import lax
from jax.experimental import pallas as pl
from jax.experimental.pallas import tpu as pltpu
from jax.experimental.pallas import tpu_sc as plsc

_N = 10000
_E = 320000
_D = 128
_NC = 2
_NS = 16
_NW = _NC * _NS
_CHUNK = 80
_NCHUNK = 128
_EPW = _NCHUNK * _CHUNK
_EPAD = _NW * _EPW - _E
_RPT = 1000
_ZROWS = 200
_CNTC = 1000
_NBUF = 2


_CW = 10016


def _make_sc_agg(with_cnt: bool):
    mesh = plsc.VectorSubcoreMesh(core_axis_name="c", subcore_axis_name="s",
                                  num_cores=_NC, num_subcores=_NS)
    out_type = [jax.ShapeDtypeStruct((_NC, _N, _D), jnp.float32)]
    if with_cnt:
        out_type.append(jax.ShapeDtypeStruct((_NW * _CW,), jnp.float32))
    _H0 = _NCHUNK // 2
    scratch = (
        [pltpu.VMEM((_CHUNK,), jnp.int32)] * _NBUF
        + [pltpu.VMEM((_H0, _CHUNK), jnp.int32)]
        + [pltpu.VMEM((_CHUNK, _D), jnp.float32)] * _NBUF
        + ([pltpu.VMEM((_CW,), jnp.float32)] if with_cnt else [])
        + [pltpu.VMEM_SHARED((_N + 8, _D), jnp.float32)]
        + [pltpu.SemaphoreType.DMA] * (3 * _NBUF + 1)
    )

    def body(src_hbm, dst_hbm, x_hbm, *refs):
        if with_cnt:
            agg_out, cnt_out = refs[0], refs[1]
            rest = refs[2:]
        else:
            agg_out = refs[0]
            rest = refs[1:]
        srcc = rest[0:_NBUF]
        dst_v = rest[_NBUF]
        rows = rest[_NBUF + 1:2 * _NBUF + 1]
        rest = rest[2 * _NBUF + 1:]
        if with_cnt:
            cnt_v = rest[0]
            rest = rest[1:]
        agg_sh = rest[0]
        sems = rest[1:]
        isem = sems[0:_NBUF]
        gsem = sems[_NBUF:2 * _NBUF]
        ssem = sems[2 * _NBUF:3 * _NBUF]
        dsem = sems[3 * _NBUF]

        cid = lax.axis_index("c")
        sid = lax.axis_index("s")
        wid = sid * _NC + cid
        base0 = wid * _EPW

        def dst_load(h):
            return pltpu.async_copy(
                dst_hbm.at[wid, pl.ds(h * _H0, _H0)], dst_v, dsem)

        def src_load(c, j):
            off = pl.multiple_of(base0 + c * _CHUNK, 8)
            pltpu.async_copy(src_hbm.at[pl.ds(off, _CHUNK)], srcc[j], isem[j])

        def src_wait(j):
            pltpu.make_async_copy(src_hbm.at[pl.ds(0, _CHUNK)], srcc[j],
                                  isem[j]).wait()

        def gather_start(c, b):
            pltpu.async_copy(x_hbm.at[srcc[b]], rows[b], gsem[b])

        def gather_wait(b):
            pltpu.make_async_copy(x_hbm.at[srcc[b]], rows[b], gsem[b]).wait()

        def scatter_start(lc, b):
            pltpu.async_copy(rows[b], agg_sh.at[dst_v.at[lc]], ssem[b],
                             add=True)

        def scatter_wait(b):
            pltpu.make_async_copy(rows[b], agg_sh.at[dst_v.at[0]],
                                  ssem[b]).wait()

        dst_cp = dst_load(0)
        src_load(0, 0)
        z16 = jnp.zeros((16,), jnp.float32)

        def fill_zrow(i, _):
            r = i // 8
            col = (i % 8) * 16
            rows[0][r, pl.ds(col, 16)] = z16
            return 0

        lax.fori_loop(0, _CHUNK * 8, fill_zrow, 0)

        if with_cnt:
            def fill_zcnt(i, _):
                cnt_v[pl.ds(i * 16, 16)] = z16
                return 0

            lax.fori_loop(0, _CW // 16, fill_zcnt, 0)

        _TAIL = _RPT - (_RPT // _CHUNK) * _CHUNK

        @pl.when(sid < _N // _RPT)
        def _():
            zcp = []
            for k in range(_RPT // _CHUNK):
                zcp.append(pltpu.async_copy(
                    rows[0], agg_sh.at[pl.ds(sid * _RPT + k * _CHUNK, _CHUNK)],
                    ssem[0]))
            zcp.append(pltpu.async_copy(
                rows[0].at[pl.ds(0, _TAIL)],
                agg_sh.at[pl.ds(sid * _RPT + (_RPT // _CHUNK) * _CHUNK,
                                _TAIL)], ssem[0]))
            for cp in zcp:
                cp.wait()
        @pl.when(sid == _N // _RPT)
        def _():
            pltpu.async_copy(rows[0].at[pl.ds(0, 8)],
                             agg_sh.at[pl.ds(_N, 8)], ssem[0]).wait()
        dst_cp.wait()
        plsc.subcore_barrier()

        one16 = jnp.ones((16,), jnp.float32)

        def count_half():
            if not with_cnt:
                return

            def cl(l, _):
                for j in range(_CHUNK // 16):
                    dvec = dst_v[l, pl.ds(j * 16, 16)]
                    plsc.addupdate_scatter(cnt_v, [dvec], one16)
                return 0

            lax.fori_loop(0, _H0, cl, 0)

        def run_half(c0):
            for b in range(_NBUF):
                src_wait(b)
                gather_start(c0 + b, b)
                if b >= 1:
                    gather_wait(b - 1)
                    src_load(c0 + b + 1, (b + 1) % _NBUF)
                    scatter_start(b - 1, b - 1)
                elif _NBUF > 1:
                    src_load(c0 + 1, 1)

            def group(g, _):
                for b in range(_NBUF):
                    l = g * _NBUF + b
                    bp = (b - 1) % _NBUF
                    bn = (b + 1) % _NBUF
                    scatter_wait(b)
                    src_wait(b)
                    gather_start(c0 + l, b)
                    gather_wait(bp)
                    @pl.when(c0 + l + 1 < _NCHUNK)
                    def _():
                        src_load(c0 + l + 1, bn)
                    scatter_start(l - 1, bp)
                return 0

            lax.fori_loop(1, _H0 // _NBUF, group, 0)

            blast = (_H0 - 1) % _NBUF
            gather_wait(blast)
            scatter_start(_H0 - 1, blast)
            for b in range(_NBUF):
                scatter_wait(b)

        run_half(0)
        count_half()
        dst_load(1).wait()
        run_half(_H0)
        count_half()

        plsc.subcore_barrier()

        ocp = []
        if with_cnt:
            ocp.append(pltpu.async_copy(
                cnt_v, cnt_out.at[pl.ds(wid * _CW, _CW)], ssem[0]))

        @pl.when(sid < _N // _RPT)
        def _():
            ocp2 = []
            for k in range(_RPT // _ZROWS):
                rs = sid * _RPT + k * _ZROWS
                ocp2.append(pltpu.async_copy(agg_sh.at[pl.ds(rs, _ZROWS)],
                                             agg_out.at[cid,
                                                        pl.ds(rs, _ZROWS)],
                                             ssem[1]))
            for cp in ocp2:
                cp.wait()
        for cp in ocp:
            cp.wait()

    return pl.kernel(body, out_type=out_type, mesh=mesh, scratch_types=scratch,
                     name="sc_sage_agg_cnt" if with_cnt else "sc_sage_agg")


_make_sc_agg = functools.lru_cache(maxsize=None)(_make_sc_agg)

def _make_sc_cnt():
    mesh = plsc.VectorSubcoreMesh(core_axis_name="c", subcore_axis_name="s",
                                  num_cores=_NC, num_subcores=_NS)
    out_type = [jax.ShapeDtypeStruct((_NC, _N, _D), jnp.float32)]
    scratch = (
        [pltpu.VMEM((_NCHUNK, _CHUNK), jnp.int32),
         pltpu.VMEM((_CHUNK, _D), jnp.float32),
         pltpu.VMEM_SHARED((_N + 8, _D), jnp.float32)]
        + [pltpu.SemaphoreType.DMA] * (_NBUF + 1)
    )

    def body(dst_hbm, cnt_out, dst_v, ones_r, cnt_sh, *sems):
        ssem = sems[0:_NBUF]
        dsem = sems[_NBUF]
        cid = lax.axis_index("c")
        sid = lax.axis_index("s")
        wid = sid * _NC + cid

        dst_cp = pltpu.async_copy(dst_hbm.at[wid], dst_v, dsem)

        z16 = jnp.zeros((16,), jnp.float32)

        def fill_zrow(i, _):
            r = i // 8
            col = (i % 8) * 16
            ones_r[r, pl.ds(col, 16)] = z16
            return 0

        lax.fori_loop(0, _CHUNK * 8, fill_zrow, 0)

        _TAIL = _RPT - (_RPT // _CHUNK) * _CHUNK

        @pl.when(sid < _N // _RPT)
        def _():
            zcp = []
            for k in range(_RPT // _CHUNK):
                zcp.append(pltpu.async_copy(
                    ones_r, cnt_sh.at[pl.ds(sid * _RPT + k * _CHUNK, _CHUNK)],
                    ssem[0]))
            zcp.append(pltpu.async_copy(
                ones_r.at[pl.ds(0, _TAIL)],
                cnt_sh.at[pl.ds(sid * _RPT + (_RPT // _CHUNK) * _CHUNK,
                                _TAIL)], ssem[0]))
            for cp in zcp:
                cp.wait()

        @pl.when(sid == _N // _RPT)
        def _():
            pltpu.async_copy(ones_r.at[pl.ds(0, 8)],
                             cnt_sh.at[pl.ds(_N, 8)], ssem[0]).wait()

        o16 = jnp.ones((16,), jnp.float32)

        def fill_ones(i, _):
            r = i // 8
            col = (i % 8) * 16
            ones_r[r, pl.ds(col, 16)] = o16
            return 0

        lax.fori_loop(0, _CHUNK * 8, fill_ones, 0)
        dst_cp.wait()
        plsc.subcore_barrier()

        def scatter_start(l, b):
            pltpu.async_copy(ones_r, cnt_sh.at[dst_v.at[l]], ssem[b],
                             add=True)

        def scatter_wait(b):
            pltpu.make_async_copy(ones_r, cnt_sh.at[dst_v.at[0]],
                                  ssem[b]).wait()

        for b in range(_NBUF):
            scatter_start(b, b)

        def group(g, _):
            for b in range(_NBUF):
                l = g * _NBUF + b
                scatter_wait(b)
                scatter_start(l, b)
            return 0

        lax.fori_loop(1, _NCHUNK // _NBUF, group, 0)
        for b in range(_NBUF):
            scatter_wait(b)
        plsc.subcore_barrier()

        @pl.when(sid < _N // _RPT)
        def _():
            ocp = []
            for k in range(_RPT // _ZROWS):
                rs = sid * _RPT + k * _ZROWS
                ocp.append(pltpu.async_copy(cnt_sh.at[pl.ds(rs, _ZROWS)],
                                            cnt_out.at[cid, pl.ds(rs, _ZROWS)],
                                            ssem[0]))
            for cp in ocp:
                cp.wait()

    return pl.kernel(body, out_type=out_type, mesh=mesh, scratch_types=scratch,
                     name="sc_sage_cnt")


_sc_cnt = None


def _get_sc_cnt():
    global _sc_cnt
    if _sc_cnt is None:
        _sc_cnt = _make_sc_cnt()
    return _sc_cnt


_BM = 1000


def _make_tc_skip():
    def body(x_ref, wr_ref, b_ref, o_ref):
        o_ref[...] = (jnp.dot(x_ref[...], wr_ref[...],
                              preferred_element_type=jnp.float32)
                      + b_ref[...])

    return pl.pallas_call(
        body,
        grid=(_N // _BM,),
        in_specs=[
            pl.BlockSpec((_BM, _D), lambda i: (i, 0)),
            pl.BlockSpec((_D, _D), lambda i: (0, 0)),
            pl.BlockSpec((1, _D), lambda i: (0, 0)),
        ],
        out_specs=pl.BlockSpec((_BM, _D), lambda i: (i, 0)),
        out_shape=jax.ShapeDtypeStruct((_N, _D), jnp.float32),
        name="tc_sage_skip",
    )


def _make_tc_combine1():
    def body(agg_ref, cnt_ref, xr_ref, wl_ref, wr2_ref, b2_ref, h_ref,
             xr2_ref):
        a = agg_ref[0] + agg_ref[1]
        c = cnt_ref[0, :, 0:1] + cnt_ref[1, :, 0:1]
        mean = a / jnp.maximum(c, 1.0)
        h = jnp.maximum(
            jnp.dot(mean, wl_ref[...], preferred_element_type=jnp.float32)
            + xr_ref[...], 0.0)
        h_ref[...] = h
        xr2_ref[...] = (jnp.dot(h, wr2_ref[...],
                                preferred_element_type=jnp.float32)
                        + b2_ref[...])

    return pl.pallas_call(
        body,
        grid=(_N // _BM,),
        in_specs=[
            pl.BlockSpec((_NC, _BM, _D), lambda i: (0, i, 0)),
            pl.BlockSpec((_NC, _BM, _D), lambda i: (0, i, 0)),
            pl.BlockSpec((_BM, _D), lambda i: (i, 0)),
            pl.BlockSpec((_D, _D), lambda i: (0, 0)),
            pl.BlockSpec((_D, _D), lambda i: (0, 0)),
            pl.BlockSpec((1, _D), lambda i: (0, 0)),
        ],
        out_specs=[pl.BlockSpec((_BM, _D), lambda i: (i, 0)),
                   pl.BlockSpec((_BM, _D), lambda i: (i, 0))],
        out_shape=[jax.ShapeDtypeStruct((_N, _D), jnp.float32),
                   jax.ShapeDtypeStruct((_N, _D), jnp.float32)],
        name="tc_sage_combine1",
    )


def _make_tc_combine2():
    def body(agg_ref, cnt_ref, xr_ref, wl_ref, o_ref):
        a = agg_ref[0] + agg_ref[1]
        c = cnt_ref[0, :, 0:1] + cnt_ref[1, :, 0:1]
        mean = a / jnp.maximum(c, 1.0)
        o_ref[...] = jnp.dot(mean, wl_ref[...],
                             preferred_element_type=jnp.float32) + xr_ref[...]

    return pl.pallas_call(
        body,
        grid=(_N // _BM,),
        in_specs=[
            pl.BlockSpec((_NC, _BM, _D), lambda i: (0, i, 0)),
            pl.BlockSpec((_NC, _BM, _D), lambda i: (0, i, 0)),
            pl.BlockSpec((_BM, _D), lambda i: (i, 0)),
            pl.BlockSpec((_D, _D), lambda i: (0, 0)),
        ],
        out_specs=pl.BlockSpec((_BM, _D), lambda i: (i, 0)),
        out_shape=jax.ShapeDtypeStruct((_N, _D), jnp.float32),
        name="tc_sage_combine2",
    )


_tc_skip = _make_tc_skip()
_tc_combine1 = _make_tc_combine1()
_tc_combine2 = _make_tc_combine2()


@jax.jit
def kernel(x, edge_index, W_l1, W_r1, b1, W_l2, W_r2, b2):
    src = jnp.pad(edge_index[0].astype(jnp.int32), (0, _EPAD + 320))
    dst = jnp.pad(edge_index[1].astype(jnp.int32), (0, _EPAD),
                  constant_values=_N)[: _NW * _EPW]
    dst = dst.reshape(_NW, _NCHUNK, _CHUNK)

    xr1 = _tc_skip(x, W_r1, b1.reshape(1, _D))
    (cntp,) = _get_sc_cnt()(dst)
    (agg1,) = _make_sc_agg(False)(src, dst, x)
    h, xr2 = _tc_combine1(agg1, cntp, xr1, W_l1, W_r2, b2.reshape(1, _D))
    (agg2,) = _make_sc_agg(False)(src, dst, h)
    out = _tc_combine2(agg2, cntp, xr2, W_l2)
    return out
